# baseline (speedup 1.0000x reference)
import functools

import jax
import jax.numpy as jnp
from jax import lax
from jax.experimental import pallas as pl
from jax.experimental.pallas import tpu as pltpu
from jax.experimental.pallas import tpu_sc as plsc

NUM_CORES = 2
NUM_SUBCORES = 16
LANES = 16
NUM_WORKERS = NUM_CORES * NUM_SUBCORES

ROWS = 32768
EXPERTS = 64
PITCH = EXPERTS + 1
ROWS_PER_W = ROWS // NUM_WORKERS
CHUNK_ROWS = 256
NUM_CHUNKS = ROWS_PER_W // CHUNK_ROWS
GROUPS = CHUNK_ROWS // LANES
STREAMS = 4
REPITCH_UNROLL = 8

_mesh = plsc.VectorSubcoreMesh(
    core_axis_name="c",
    subcore_axis_name="s",
    num_cores=NUM_CORES,
    num_subcores=NUM_SUBCORES,
)


@functools.partial(
    pl.kernel,
    out_type=(
        jax.ShapeDtypeStruct((ROWS * 2 // 128, 128), jnp.float32),
        jax.ShapeDtypeStruct((ROWS * 2 // 128, 128), jnp.int32),
    ),
    mesh=_mesh,
    scratch_types=(
        pltpu.VMEM((CHUNK_ROWS, EXPERTS), jnp.float32),
        pltpu.VMEM((CHUNK_ROWS * PITCH,), jnp.float32),
        pltpu.VMEM((ROWS_PER_W * 2 // 128, 128), jnp.float32),
        pltpu.VMEM((ROWS_PER_W * 2 // 128, 128), jnp.int32),
    ),
    compiler_params=pltpu.CompilerParams(
        needs_layout_passes=False, use_tc_tiling_on_sc=True
    ),
)
def _router(logits_hbm, w_hbm, ids_hbm, raw_v, in_v, w_v, ids_v):
    wid = lax.axis_index("s") * NUM_CORES + lax.axis_index("c")
    base = wid * ROWS_PER_W

    lane = lax.iota(jnp.int32, LANES)
    zero = jnp.zeros((LANES,), jnp.int32)
    one = jnp.ones((LANES,), jnp.int32)

    def chunk_body(ch, carry):
        row0 = base + ch * CHUNK_ROWS
        pltpu.sync_copy(logits_hbm.at[pl.ds(row0, CHUNK_ROWS)], raw_v)

        def repitch_body(rr, carry_in):
            r0 = rr * REPITCH_UNROLL
            vals = []
            for u in range(REPITCH_UNROLL):
                for j in range(EXPERTS // LANES):
                    vals.append(raw_v[r0 + u, pl.ds(j * LANES, LANES)])
            k = 0
            for u in range(REPITCH_UNROLL):
                for j in range(EXPERTS // LANES):
                    in_v[pl.ds((r0 + u) * PITCH + j * LANES, LANES)] = vals[k]
                    k += 1
            return carry_in

        lax.fori_loop(0, CHUNK_ROWS // REPITCH_UNROLL, repitch_body, 0)

        def group_body(gs, carry_in):
            row_idx, flat0 = [], []
            m1, i1, m2, i2 = [], [], [], []
            neg_inf = jnp.full((LANES,), -jnp.inf, jnp.float32)
            for t in range(STREAMS):
                r = (gs * STREAMS + t) * LANES + lane
                row_idx.append(r)
                flat0.append(r * PITCH)
                m1.append(plsc.load_gather(in_v, [flat0[t]]))
                i1.append(zero)
                m2.append(neg_inf)
                i2.append(zero)
            for e in range(1, EXPERTS):
                e_vec = jnp.full((LANES,), e, jnp.int32)
                for t in range(STREAMS):
                    v = plsc.load_gather(in_v, [flat0[t] + e])
                    gt1 = v > m1[t]
                    gt2 = v > m2[t]
                    m2[t] = jnp.where(gt1, m1[t], jnp.where(gt2, v, m2[t]))
                    i2[t] = jnp.where(gt1, i1[t], jnp.where(gt2, e_vec, i2[t]))
                    m1[t] = jnp.where(gt1, v, m1[t])
                    i1[t] = jnp.where(gt1, e_vec, i1[t])
            for t in range(STREAMS):
                ex = jnp.exp(m2[t] - m1[t])
                s = 1.0 + ex
                w1 = 1.0 / s
                w2 = ex / s
                f0 = ch * (CHUNK_ROWS * 2) + row_idx[t] * 2
                f1 = f0 + 1
                r0v, c0v = f0 >> 7, f0 & 127
                r1v, c1v = f1 >> 7, f1 & 127
                plsc.store_scatter(w_v, [r0v, c0v], w1)
                plsc.store_scatter(w_v, [r1v, c1v], w2)
                plsc.store_scatter(ids_v, [r0v, c0v], i1[t])
                plsc.store_scatter(ids_v, [r1v, c1v], i2[t])
            return carry_in

        lax.fori_loop(0, GROUPS // STREAMS, group_body, 0)
        return carry

    lax.fori_loop(0, NUM_CHUNKS, chunk_body, 0)
    blk = ROWS_PER_W * 2 // 128
    pltpu.sync_copy(w_v, w_hbm.at[pl.ds(wid * blk, blk)])
    pltpu.sync_copy(ids_v, ids_hbm.at[pl.ds(wid * blk, blk)])


def kernel(router_logits):
    w_blk, ids_blk = _router(router_logits)
    topk_weights = w_blk.reshape(ROWS, 2)
    topk_ids = ids_blk.reshape(ROWS, 2)
    return (topk_weights, topk_ids, router_logits)


# speedup vs baseline: 1.2901x; 1.2901x over previous
import functools

import jax
import jax.numpy as jnp
from jax import lax
from jax.experimental import pallas as pl
from jax.experimental.pallas import tpu as pltpu
from jax.experimental.pallas import tpu_sc as plsc

NUM_CORES = 2
NUM_SUBCORES = 16
LANES = 16
NUM_WORKERS = NUM_CORES * NUM_SUBCORES

ROWS = 32768
EXPERTS = 64
PITCH = EXPERTS + 1
ROWS_PER_W = ROWS // NUM_WORKERS
CHUNK_ROWS = 256
NUM_CHUNKS = ROWS_PER_W // CHUNK_ROWS
GROUPS = CHUNK_ROWS // LANES
STREAMS = 4
REPITCH_UNROLL = 8

_mesh = plsc.VectorSubcoreMesh(
    core_axis_name="c",
    subcore_axis_name="s",
    num_cores=NUM_CORES,
    num_subcores=NUM_SUBCORES,
)


@functools.partial(
    pl.kernel,
    out_type=(
        jax.ShapeDtypeStruct((ROWS, 2), jnp.float32),
        jax.ShapeDtypeStruct((ROWS, 2), jnp.int32),
    ),
    mesh=_mesh,
    scratch_types=(
        pltpu.VMEM((CHUNK_ROWS, EXPERTS), jnp.float32),
        pltpu.VMEM((CHUNK_ROWS * PITCH,), jnp.float32),
        pltpu.VMEM((CHUNK_ROWS, 2), jnp.float32),
        pltpu.VMEM((CHUNK_ROWS, 2), jnp.int32),
    ),
    compiler_params=pltpu.CompilerParams(
        needs_layout_passes=False, use_tc_tiling_on_sc=True
    ),
)
def _router(logits_hbm, w_hbm, ids_hbm, raw_v, in_v, w_v, ids_v):
    wid = lax.axis_index("s") * NUM_CORES + lax.axis_index("c")
    base = wid * ROWS_PER_W

    lane = lax.iota(jnp.int32, LANES)
    zero = jnp.zeros((LANES,), jnp.int32)
    one = jnp.ones((LANES,), jnp.int32)

    def chunk_body(ch, carry):
        row0 = base + ch * CHUNK_ROWS
        pltpu.sync_copy(logits_hbm.at[pl.ds(row0, CHUNK_ROWS)], raw_v)

        def repitch_body(rr, carry_in):
            r0 = rr * REPITCH_UNROLL
            vals = []
            for u in range(REPITCH_UNROLL):
                for j in range(EXPERTS // LANES):
                    vals.append(raw_v[r0 + u, pl.ds(j * LANES, LANES)])
            k = 0
            for u in range(REPITCH_UNROLL):
                for j in range(EXPERTS // LANES):
                    in_v[pl.ds((r0 + u) * PITCH + j * LANES, LANES)] = vals[k]
                    k += 1
            return carry_in

        lax.fori_loop(0, CHUNK_ROWS // REPITCH_UNROLL, repitch_body, 0)

        def group_body(gs, carry_in):
            row_idx, flat0 = [], []
            m1, i1, m2, i2 = [], [], [], []
            neg_inf = jnp.full((LANES,), -jnp.inf, jnp.float32)
            for t in range(STREAMS):
                r = (gs * STREAMS + t) * LANES + lane
                row_idx.append(r)
                flat0.append(r * PITCH)
                m1.append(plsc.load_gather(in_v, [flat0[t]]))
                i1.append(zero)
                m2.append(neg_inf)
                i2.append(zero)
            for e in range(1, EXPERTS):
                e_vec = jnp.full((LANES,), e, jnp.int32)
                for t in range(STREAMS):
                    v = plsc.load_gather(in_v, [flat0[t] + e])
                    gt1 = v > m1[t]
                    gt2 = v > m2[t]
                    m2[t] = jnp.where(gt1, m1[t], jnp.where(gt2, v, m2[t]))
                    i2[t] = jnp.where(gt1, i1[t], jnp.where(gt2, e_vec, i2[t]))
                    m1[t] = jnp.where(gt1, v, m1[t])
                    i1[t] = jnp.where(gt1, e_vec, i1[t])
            for t in range(STREAMS):
                ex = jnp.exp(m2[t] - m1[t])
                s = 1.0 + ex
                w1 = 1.0 / s
                w2 = ex / s
                plsc.store_scatter(w_v, [row_idx[t], zero], w1)
                plsc.store_scatter(w_v, [row_idx[t], one], w2)
                plsc.store_scatter(ids_v, [row_idx[t], zero], i1[t])
                plsc.store_scatter(ids_v, [row_idx[t], one], i2[t])
            return carry_in

        lax.fori_loop(0, GROUPS // STREAMS, group_body, 0)
        pltpu.sync_copy(w_v, w_hbm.at[pl.ds(row0, CHUNK_ROWS)])
        pltpu.sync_copy(ids_v, ids_hbm.at[pl.ds(row0, CHUNK_ROWS)])
        return carry

    lax.fori_loop(0, NUM_CHUNKS, chunk_body, 0)


def kernel(router_logits):
    topk_weights, topk_ids = _router(router_logits)
    return (topk_weights, topk_ids, router_logits)
